# baseline (speedup 1.0000x reference)
import functools

import jax
import jax.numpy as jnp
from jax import lax
from jax.experimental import pallas as pl
from jax.experimental.pallas import tpu as pltpu
from jax.experimental.pallas import tpu_sc as plsc

_V = 1000
_VPAD = 1024
_N = 1024 * 50
_NC = 2
_NS = 16
_NW = _NC * _NS
_PER_W = _N // _NW
_CH = 32
_NCHUNK = _PER_W // _CH


def _lse_body(table_ref, lse_ref):
    t = table_ref[...]
    m = jnp.max(t, axis=1)
    s = jnp.sum(jnp.exp(t - m[:, None]), axis=1)
    lse = m + jnp.log(s)
    lse_ref[...] = jnp.concatenate(
        [lse, jnp.zeros((_VPAD - _V,), jnp.float32)])


_lse_call = pl.pallas_call(
    _lse_body,
    out_shape=jax.ShapeDtypeStruct((_VPAD,), jnp.float32),
)

_mesh = plsc.VectorSubcoreMesh(
    core_axis_name="c", subcore_axis_name="s",
    num_cores=_NC, num_subcores=_NS)


@functools.partial(
    pl.kernel,
    mesh=_mesh,
    compiler_params=pltpu.CompilerParams(
        needs_layout_passes=False, use_tc_tiling_on_sc=True),
    out_type=[
        jax.ShapeDtypeStruct((_N, _V), jnp.float32),
        jax.ShapeDtypeStruct((_N, 128), jnp.float32),
        jax.ShapeDtypeStruct((_NW, 16), jnp.float32),
    ],
    scratch_types=[
        pltpu.VMEM((_CH,), jnp.int32),
        pltpu.VMEM((_CH,), jnp.int32),
        pltpu.VMEM((_CH,), jnp.int32),
        pltpu.VMEM((_CH,), jnp.float32),
        pltpu.VMEM((_CH, _VPAD), jnp.float32),
        pltpu.VMEM((_VPAD,), jnp.float32),
        pltpu.VMEM((16,), jnp.float32),
        pltpu.SemaphoreType.DMA,
        pltpu.SemaphoreType.DMA,
    ],
)
def _sc_main(table_hbm, tflat_hbm, idx_hbm, tgt_hbm, lse_hbm,
             out_hbm, strip_hbm, part_hbm,
             idx_v, tgt_v, fidx_v, pick_v, rows_v, lse_v, acc_v,
             sem, sem2):
    wid = lax.axis_index("s") * _NC + lax.axis_index("c")
    base0 = wid * _PER_W
    pltpu.sync_copy(lse_hbm, lse_v)

    def chunk(c, acc):
        base = base0 + c * _CH
        pltpu.sync_copy(idx_hbm.at[pl.ds(base, _CH)], idx_v)
        pltpu.sync_copy(tgt_hbm.at[pl.ds(base, _CH)], tgt_v)
        row_dma = pltpu.async_copy(table_hbm.at[idx_v], rows_v, sem)
        for k in range(_CH // 16):
            ii = idx_v[pl.ds(k * 16, 16)]
            tt = tgt_v[pl.ds(k * 16, 16)]
            fidx_v[pl.ds(k * 16, 16)] = ii * _VPAD + tt
        pick_dma = pltpu.async_copy(tflat_hbm.at[fidx_v], pick_v, sem2)
        row_dma.wait()
        pltpu.sync_copy(rows_v.at[:, pl.ds(0, 896)],
                        out_hbm.at[pl.ds(base, _CH), pl.ds(0, 896)])
        pltpu.sync_copy(rows_v.at[:, pl.ds(896, 128)],
                        strip_hbm.at[pl.ds(base, _CH)])
        pick_dma.wait()
        for k in range(_CH // 16):
            ii = idx_v[pl.ds(k * 16, 16)]
            lse_vals = plsc.load_gather(lse_v, [ii])
            acc = acc + lse_vals - pick_v[pl.ds(k * 16, 16)]
        return acc

    acc = lax.fori_loop(0, _NCHUNK, chunk, jnp.zeros((16,), jnp.float32))
    acc_v[...] = acc
    pltpu.sync_copy(acc_v, part_hbm.at[wid])


def kernel(idx, targets, token_embedding_table):
    idx_f = idx.reshape(-1)
    tgt_f = targets.reshape(-1)
    table_pad = jnp.pad(token_embedding_table, ((0, 0), (0, _VPAD - _V)))
    table_flat = table_pad.reshape(-1)
    lse = _lse_call(token_embedding_table)
    out_main, strip, parts = _sc_main(table_pad, table_flat, idx_f, tgt_f, lse)
    flat_logits = lax.dynamic_update_slice(
        out_main, strip[:, :_V - 896], (0, 896))
    loss = jnp.sum(parts) / jnp.float32(_N)
    return (flat_logits, loss)


# speedup vs baseline: 1.2822x; 1.2822x over previous
import functools

import jax
import jax.numpy as jnp
from jax import lax
from jax.experimental import pallas as pl
from jax.experimental.pallas import tpu as pltpu
from jax.experimental.pallas import tpu_sc as plsc

_V = 1000
_VPAD = 1024
_N = 1024 * 50
_NC = 2
_NS = 16
_NW = _NC * _NS
_PER_W = _N // _NW
_CH = 32
_NCHUNK = _PER_W // _CH


def _lse_body(table_ref, lse_ref):
    t = table_ref[...]
    m = jnp.max(t, axis=1)
    s = jnp.sum(jnp.exp(t - m[:, None]), axis=1)
    lse = m + jnp.log(s)
    lse_ref[...] = jnp.concatenate(
        [lse, jnp.zeros((_VPAD - _V,), jnp.float32)])


_lse_call = pl.pallas_call(
    _lse_body,
    out_shape=jax.ShapeDtypeStruct((_VPAD,), jnp.float32),
)

_mesh = plsc.VectorSubcoreMesh(
    core_axis_name="c", subcore_axis_name="s",
    num_cores=_NC, num_subcores=_NS)


@functools.partial(
    pl.kernel,
    mesh=_mesh,
    compiler_params=pltpu.CompilerParams(
        needs_layout_passes=False, use_tc_tiling_on_sc=True),
    out_type=[
        jax.ShapeDtypeStruct((_N, _VPAD), jnp.float32),
        jax.ShapeDtypeStruct((_NW, 16), jnp.float32),
    ],
    scratch_types=[
        pltpu.VMEM((_PER_W,), jnp.int32),
        pltpu.VMEM((_PER_W,), jnp.int32),
        pltpu.VMEM((_PER_W,), jnp.int32),
        pltpu.VMEM((_PER_W,), jnp.float32),
        pltpu.VMEM((_CH, _VPAD), jnp.float32),
        pltpu.VMEM((_CH, _VPAD), jnp.float32),
        pltpu.VMEM((_VPAD,), jnp.float32),
        pltpu.VMEM((16,), jnp.float32),
        pltpu.SemaphoreType.DMA,
        pltpu.SemaphoreType.DMA,
        pltpu.SemaphoreType.DMA,
    ],
)
def _sc_main(table_hbm, tflat_hbm, idx_hbm, tgt_hbm, lse_hbm,
             out_hbm, part_hbm,
             idx_a, tgt_a, fidx_a, pick_a, rows0, rows1, lse_v, acc_v,
             sem0, sem1, semp):
    wid = lax.axis_index("s") * _NC + lax.axis_index("c")
    base0 = wid * _PER_W
    pltpu.sync_copy(idx_hbm.at[pl.ds(base0, _PER_W)], idx_a)
    pltpu.sync_copy(tgt_hbm.at[pl.ds(base0, _PER_W)], tgt_a)
    pltpu.sync_copy(lse_hbm, lse_v)

    def build_fidx(k, carry):
        ii = idx_a[pl.ds(k * 16, 16)]
        tt = tgt_a[pl.ds(k * 16, 16)]
        fidx_a[pl.ds(k * 16, 16)] = ii * _VPAD + tt
        return carry

    lax.fori_loop(0, _PER_W // 16, build_fidx, 0)
    pick_dma = pltpu.async_copy(tflat_hbm.at[fidx_a], pick_a, semp)

    rows = (rows0, rows1)
    sems = (sem0, sem1)

    pltpu.async_copy(table_hbm.at[idx_a.at[pl.ds(0, _CH)]], rows0, sem0)

    def pair(j, carry):
        for p in range(2):
            c = j * 2 + p
            nxt = 1 - p

            @pl.when(c + 1 < _NCHUNK)
            def _issue_next():
                pltpu.async_copy(
                    table_hbm.at[idx_a.at[pl.ds((c + 1) * _CH, _CH)]],
                    rows[nxt], sems[nxt])

            pltpu.make_async_copy(
                table_hbm.at[pl.ds(0, _CH)], rows[p], sems[p]).wait()
            pltpu.sync_copy(rows[p],
                            out_hbm.at[pl.ds(base0 + c * _CH, _CH)])
        return carry

    lax.fori_loop(0, _NCHUNK // 2, pair, 0)

    pick_dma.wait()

    def accum(k, acc):
        ii = idx_a[pl.ds(k * 16, 16)]
        lse_vals = plsc.load_gather(lse_v, [ii])
        return acc + lse_vals - pick_a[pl.ds(k * 16, 16)]

    acc = lax.fori_loop(0, _PER_W // 16, accum,
                        jnp.zeros((16,), jnp.float32))
    acc_v[...] = acc
    pltpu.sync_copy(acc_v, part_hbm.at[wid])


def kernel(idx, targets, token_embedding_table):
    idx_f = idx.reshape(-1)
    tgt_f = targets.reshape(-1)
    table_pad = jnp.pad(token_embedding_table, ((0, 0), (0, _VPAD - _V)))
    table_flat = table_pad.reshape(-1)
    lse = _lse_call(token_embedding_table)
    out_pad, parts = _sc_main(table_pad, table_flat, idx_f, tgt_f, lse)
    flat_logits = out_pad[:, :_V]
    loss = jnp.sum(parts) / jnp.float32(_N)
    return (flat_logits, loss)
